# Initial kernel scaffold; baseline (speedup 1.0000x reference)
import functools

import jax
import jax.numpy as jnp
from jax import lax
from jax.experimental import pallas as pl
from jax.experimental.pallas import tpu as pltpu

E = 64
K = 8
COEF = 0.01
TB = 1024


def _body(x_ref, w_ref, tw_ref, ti_ref, aux_ref, cnt_ref, ps_ref):
    i = pl.program_id(0)
    n = pl.num_programs(0)
    x = x_ref[...]
    w = w_ref[...]
    logits = lax.dot_general(x, w, (((1,), (1,)), ((), ())),
                             preferred_element_type=jnp.float32)

    @pl.when(i == 0)
    def _init():
        cnt_ref[...] = jnp.zeros_like(cnt_ref)
        ps_ref[...] = jnp.zeros_like(ps_ref)

    rowmax = jnp.max(logits, axis=1, keepdims=True)
    ex = jnp.exp(logits - rowmax)
    probs = ex / jnp.sum(ex, axis=1, keepdims=True)
    ps_ref[...] = ps_ref[...] + jnp.sum(probs, axis=0, keepdims=True)

    iota = lax.broadcasted_iota(jnp.int32, (TB, E), 1)
    l = logits
    vals = []
    idxs = []
    cnt = jnp.zeros((1, E), jnp.float32)
    for _ in range(K):
        m = jnp.max(l, axis=1, keepdims=True)
        cand = jnp.where(l == m, iota, E)
        am = jnp.min(cand, axis=1, keepdims=True)
        sel = iota == am
        cnt = cnt + jnp.sum(sel.astype(jnp.float32), axis=0, keepdims=True)
        vals.append(m)
        idxs.append(am)
        l = jnp.where(sel, jnp.float32(-1e30), l)
    tv = jnp.concatenate(vals, axis=1)
    ti = jnp.concatenate(idxs, axis=1)
    ex2 = jnp.exp(tv - tv[:, :1])
    tw_ref[...] = ex2 / jnp.sum(ex2, axis=1, keepdims=True)
    ti_ref[...] = ti
    cnt_ref[...] = cnt_ref[...] + cnt

    @pl.when(i == n - 1)
    def _fin():
        t = n * TB
        aux = jnp.sum(cnt_ref[...] * ps_ref[...]) * (COEF * E / (t * t))
        aux_ref[0, 0] = aux


@jax.jit
def kernel(hidden_states, gate_w):
    t, h = hidden_states.shape
    grid = t // TB
    tw, ti, aux = pl.pallas_call(
        _body,
        grid=(grid,),
        in_specs=[pl.BlockSpec((TB, h), lambda i: (i, 0)),
                  pl.BlockSpec((E, h), lambda i: (0, 0))],
        out_specs=[pl.BlockSpec((TB, K), lambda i: (i, 0)),
                   pl.BlockSpec((TB, K), lambda i: (i, 0)),
                   pl.BlockSpec((1, 1), lambda i: (0, 0))],
        out_shape=[jax.ShapeDtypeStruct((t, K), jnp.float32),
                   jax.ShapeDtypeStruct((t, K), jnp.int32),
                   jax.ShapeDtypeStruct((1, 1), jnp.float32)],
        scratch_shapes=[pltpu.VMEM((1, E), jnp.float32),
                        pltpu.VMEM((1, E), jnp.float32)],
    )(hidden_states, gate_w)
    return tw, ti, aux[0, 0]


# speedup vs baseline: 1.3268x; 1.3268x over previous
import functools

import jax
import jax.numpy as jnp
from jax import lax
from jax.experimental import pallas as pl
from jax.experimental.pallas import tpu as pltpu

E = 64
K = 8
COEF = 0.01
TB = 1024


def _body(x_ref, w_ref, tw_ref, ti_ref, aux_ref, cnt_ref, ps_ref):
    i = pl.program_id(0)
    n = pl.num_programs(0)
    x = x_ref[...]
    w = w_ref[...]
    logits = lax.dot_general(x, w, (((1,), (1,)), ((), ())),
                             preferred_element_type=jnp.float32)

    @pl.when(i == 0)
    def _init():
        cnt_ref[...] = jnp.zeros_like(cnt_ref)
        ps_ref[...] = jnp.zeros_like(ps_ref)

    rowmax = jnp.max(logits, axis=1, keepdims=True)
    ex = jnp.exp(logits - rowmax)
    probs = ex / jnp.sum(ex, axis=1, keepdims=True)
    ps_ref[...] = ps_ref[...] + jnp.sum(probs, axis=0, keepdims=True)

    iota = lax.broadcasted_iota(jnp.int32, (TB, E), 1)
    l = logits
    vals = []
    idxs = []
    cnt = jnp.zeros((1, E), jnp.float32)
    for _ in range(K):
        m = jnp.max(l, axis=1, keepdims=True)
        cand = jnp.where(l == m, iota, E)
        am = jnp.min(cand, axis=1, keepdims=True)
        sel = iota == am
        cnt = cnt + jnp.sum(sel.astype(jnp.float32), axis=0, keepdims=True)
        vals.append(m)
        idxs.append(am)
        l = jnp.where(sel, jnp.float32(-1e30), l)
    tv = jnp.concatenate(vals, axis=1)
    ti = jnp.concatenate(idxs, axis=1)
    ex2 = jnp.exp(tv - tv[:, :1])
    tw_ref[...] = ex2 / jnp.sum(ex2, axis=1, keepdims=True)
    ti_ref[...] = ti
    cnt_ref[...] = cnt_ref[...] + cnt

    @pl.when(i == n - 1)
    def _fin():
        t = n * TB
        aux = jnp.sum(cnt_ref[...] * ps_ref[...]) * (COEF * E / (t * t))
        aux_ref[0, 0] = aux


@jax.jit
def kernel(hidden_states, gate_w):
    t, h = hidden_states.shape
    grid = t // TB
    tw, ti, aux = pl.pallas_call(
        _body,
        grid=(grid,),
        in_specs=[pl.BlockSpec((TB, h), lambda i: (i, 0)),
                  pl.BlockSpec((E, h), lambda i: (0, 0))],
        out_specs=[pl.BlockSpec((TB, K), lambda i: (i, 0)),
                   pl.BlockSpec((TB, K), lambda i: (i, 0)),
                   pl.BlockSpec((1, 1), lambda i: (0, 0),
                                memory_space=pltpu.SMEM)],
        out_shape=[jax.ShapeDtypeStruct((t, K), jnp.float32),
                   jax.ShapeDtypeStruct((t, K), jnp.int32),
                   jax.ShapeDtypeStruct((1, 1), jnp.float32)],
        scratch_shapes=[pltpu.VMEM((1, E), jnp.float32),
                        pltpu.VMEM((1, E), jnp.float32)],
    )(hidden_states, gate_w)
    return tw, ti, aux[0, 0]


# speedup vs baseline: 1.3863x; 1.0449x over previous
import functools

import jax
import jax.numpy as jnp
from jax import lax
from jax.experimental import pallas as pl
from jax.experimental.pallas import tpu as pltpu
from jax.experimental.pallas import tpu_sc as plsc

E = 64
K = 8
COEF = 0.01
TB = 1024
NC, NS, L = 2, 16, 16
NW = NC * NS
MASK6 = ~63
SENT = -2147483648


def _tc_body(x_ref, w_ref, lt_ref, s_ref, ps_ref):
    i = pl.program_id(0)
    n = pl.num_programs(0)
    x = x_ref[...]
    w = w_ref[...]
    lt = lax.dot_general(w, x, (((1,), (1,)), ((), ())),
                         preferred_element_type=jnp.float32)

    @pl.when(i == 0)
    def _init():
        ps_ref[...] = jnp.zeros_like(ps_ref)

    colmax = jnp.max(lt, axis=0, keepdims=True)
    ex = jnp.exp(lt - colmax)
    probs = ex / jnp.sum(ex, axis=0, keepdims=True)
    ps_ref[...] = ps_ref[...] + jnp.sum(probs, axis=1, keepdims=True)
    lt_ref[...] = lt[None]

    @pl.when(i == n - 1)
    def _fin():
        t = n * TB
        s_ref[...] = ps_ref[...] * (COEF * E / (t * t))


def _ord(u):
    return u ^ (lax.shift_right_arithmetic(u, 31) & 0x7FFFFFFF)


def _sc_body(lt_hbm, s_hbm, tw_hbm, ti_hbm, auxp_hbm,
             lt_v, s_v, tw_v, ti_v, aux_v):
    c = lax.axis_index("c")
    sx = lax.axis_index("s")
    wid = sx * NC + c

    pltpu.sync_copy(lt_hbm.at[wid], lt_v)
    pltpu.sync_copy(s_hbm, s_v)

    s_tab = [s_v[pl.ds(16 * p, L)] for p in range(4)]

    def group(g, acc):
        base = g * L
        t = [jnp.full((L,), SENT, jnp.int32) for _ in range(K)]
        for e in range(E):
            v = lt_v[pl.ds(e * TB + base, L)]
            u = lax.bitcast_convert_type(v, jnp.int32)
            cur = (_ord(u) & MASK6) | (63 - e)
            for j in range(K):
                hi = lax.max(t[j], cur)
                cur = lax.min(t[j], cur)
                t[j] = hi
        idxs = []
        exs = []
        v0 = None
        for j in range(K):
            aj = 63 - (t[j] & 63)
            vj = lax.bitcast_convert_type(_ord(t[j] & MASK6), jnp.float32)
            if j == 0:
                v0 = vj
            idxs.append(aj)
            exs.append(jnp.exp(vj - v0))
            p = lax.shift_right_logical(aj, 4)
            wi = aj & 15
            gv = jnp.take(s_tab[3], wi, mode="fill")
            for q in range(2, -1, -1):
                gv = jnp.where(p == q, jnp.take(s_tab[q], wi, mode="fill"), gv)
            acc = acc + gv
        tot = exs[0]
        for j in range(1, K):
            tot = tot + exs[j]
        inv = 1.0 / tot
        for j in range(K):
            tw_v[pl.ds(j * TB + base, L)] = exs[j] * inv
            ti_v[pl.ds(j * TB + base, L)] = idxs[j]
        return acc

    acc = lax.fori_loop(0, TB // L, group, jnp.zeros((L,), jnp.float32))
    aux_v[...] = acc

    t_total = NW * TB
    for j in range(K):
        pltpu.sync_copy(tw_v.at[pl.ds(j * TB, TB)],
                        tw_hbm.at[pl.ds(j * t_total + wid * TB, TB)])
        pltpu.sync_copy(ti_v.at[pl.ds(j * TB, TB)],
                        ti_hbm.at[pl.ds(j * t_total + wid * TB, TB)])
    pltpu.sync_copy(aux_v, auxp_hbm.at[wid])


@jax.jit
def kernel(hidden_states, gate_w):
    t, h = hidden_states.shape
    grid = t // TB
    lt3, s2 = pl.pallas_call(
        _tc_body,
        grid=(grid,),
        in_specs=[pl.BlockSpec((TB, h), lambda i: (i, 0)),
                  pl.BlockSpec((E, h), lambda i: (0, 0))],
        out_specs=[pl.BlockSpec((1, E, TB), lambda i: (i, 0, 0)),
                   pl.BlockSpec((E, 1), lambda i: (0, 0))],
        out_shape=[jax.ShapeDtypeStruct((grid, E, TB), jnp.float32),
                   jax.ShapeDtypeStruct((E, 1), jnp.float32)],
        scratch_shapes=[pltpu.VMEM((E, 1), jnp.float32)],
    )(hidden_states, gate_w)

    mesh = plsc.VectorSubcoreMesh(core_axis_name="c", subcore_axis_name="s")
    sc = functools.partial(
        pl.kernel,
        mesh=mesh,
        compiler_params=pltpu.CompilerParams(use_tc_tiling_on_sc=False),
        out_type=[jax.ShapeDtypeStruct((K * t,), jnp.float32),
                  jax.ShapeDtypeStruct((K * t,), jnp.int32),
                  jax.ShapeDtypeStruct((NW, L), jnp.float32)],
        scratch_types=[pltpu.VMEM((E * TB,), jnp.float32),
                       pltpu.VMEM((E,), jnp.float32),
                       pltpu.VMEM((K * TB,), jnp.float32),
                       pltpu.VMEM((K * TB,), jnp.int32),
                       pltpu.VMEM((L,), jnp.float32)],
    )(_sc_body)
    twf, tif, auxp = sc(lt3.reshape(grid, E * TB), s2.reshape(E))
    tw = twf.reshape(K, t).T
    ti = tif.reshape(K, t).T
    return tw, ti, jnp.sum(auxp)
